# baseline (speedup 1.0000x reference)
import functools

import jax
import jax.numpy as jnp
from jax import lax
from jax.experimental import pallas as pl
from jax.experimental.pallas import tpu as pltpu
from jax.experimental.pallas import tpu_sc as plsc

N = 10000
E = 320000
D_IN = 128
H = 64
HH = H // 2

NC = 2
NS = 16
NW = NC * NS

CHUNK = 256
E_PAD = 327680
N_PAD = 10016
CPT = E_PAD // CHUNK // NW
BATCH = 8
NBATCH = CPT // BATCH
ROWS_PER_TILE = N_PAD // NS
DEG_W = 16

_mesh = plsc.VectorSubcoreMesh(core_axis_name="c", subcore_axis_name="s")
_params = pltpu.CompilerParams(use_tc_tiling_on_sc=False)
_tc_params = pltpu.CompilerParams(vmem_limit_bytes=100 * 1024 * 1024)


def _worker_id():
    return lax.axis_index("s") * NC + lax.axis_index("c")


def _zero_fill(buf, nrows, ncols):
    zeros = jnp.zeros((16,), jnp.float32)

    def body(r, _):
        for c in range(ncols // 16):
            buf[r, pl.ds(c * 16, 16)] = zeros
        return 0

    lax.fori_loop(0, nrows, body, 0)


@functools.partial(
    pl.kernel,
    mesh=_mesh,
    compiler_params=_params,
    out_type=jax.ShapeDtypeStruct((NC, N_PAD, DEG_W), jnp.float32),
    scratch_types=[
        pltpu.VMEM((CPT, CHUNK), jnp.int32),
        pltpu.VMEM((CHUNK, DEG_W), jnp.float32),
        pltpu.VMEM((ROWS_PER_TILE, DEG_W), jnp.float32),
        pltpu.VMEM_SHARED((N_PAD, DEG_W), jnp.float32),
        pltpu.SemaphoreType.DMA,
    ],
)
def _sc_degree(col_hbm, out_hbm, cidx_v, ones_v, stage_v, acc_sh, sem):
    cid = lax.axis_index("c")
    sid = lax.axis_index("s")
    wid = _worker_id()

    ones = jnp.ones((16,), jnp.float32)

    def fill_ones(r, _):
        ones_v[r, pl.ds(0, 16)] = ones
        return 0

    lax.fori_loop(0, CHUNK, fill_ones, 0)

    _zero_fill(stage_v, ROWS_PER_TILE, DEG_W)
    row0 = sid * ROWS_PER_TILE
    pltpu.sync_copy(stage_v, acc_sh.at[pl.ds(row0, ROWS_PER_TILE), :])
    plsc.subcore_barrier()

    chunk0 = pl.multiple_of(wid * CPT, BATCH)
    pltpu.sync_copy(col_hbm.at[pl.ds(chunk0, CPT), :], cidx_v)

    def batch_body(b, _):
        j0 = b * BATCH
        handles = []
        for j in range(BATCH):
            handles.append(
                pltpu.async_copy(ones_v, acc_sh.at[cidx_v.at[j0 + j]], sem,
                                 add=True))
        for h in handles:
            h.wait()
        return 0

    lax.fori_loop(0, NBATCH, batch_body, 0)
    plsc.subcore_barrier()

    pltpu.sync_copy(acc_sh.at[pl.ds(row0, ROWS_PER_TILE), :], stage_v)
    pltpu.sync_copy(stage_v, out_hbm.at[cid, pl.ds(row0, ROWS_PER_TILE), :])


@functools.partial(
    pl.kernel,
    mesh=_mesh,
    compiler_params=_params,
    out_type=jax.ShapeDtypeStruct((NC, 2, N_PAD, HH), jnp.float32),
    scratch_types=[
        pltpu.VMEM((CPT, CHUNK), jnp.int32),
        pltpu.VMEM((CPT, CHUNK), jnp.int32),
        pltpu.VMEM((4, CHUNK, HH), jnp.float32),
        pltpu.VMEM((ROWS_PER_TILE, HH), jnp.float32),
        pltpu.VMEM_SHARED((N_PAD, HH), jnp.float32),
        pltpu.VMEM_SHARED((N_PAD, HH), jnp.float32),
        pltpu.SemaphoreType.DMA((4,)),
        pltpu.SemaphoreType.DMA((4,)),
    ],
)
def _sc_agg(yl_hbm, yr_hbm, row_hbm, col_hbm, out_hbm,
            ridx_v, cidx_v, rows_v, stage_v, acc_sh, y_sh, gsem, ssem):
    cid = lax.axis_index("c")
    sid = lax.axis_index("s")
    wid = _worker_id()

    row0 = sid * ROWS_PER_TILE
    chunk0 = pl.multiple_of(wid * CPT, BATCH)

    pltpu.sync_copy(row_hbm.at[pl.ds(chunk0, CPT), :], ridx_v)
    pltpu.sync_copy(col_hbm.at[pl.ds(chunk0, CPT), :], cidx_v)

    for h, y_hbm in enumerate((yl_hbm, yr_hbm)):
        pltpu.sync_copy(y_hbm.at[pl.ds(row0, ROWS_PER_TILE), :], stage_v)
        pltpu.sync_copy(stage_v, y_sh.at[pl.ds(row0, ROWS_PER_TILE), :])
        _zero_fill(stage_v, ROWS_PER_TILE, HH)
        pltpu.sync_copy(stage_v, acc_sh.at[pl.ds(row0, ROWS_PER_TILE), :])
        plsc.subcore_barrier()

        def batch_body(b, _):
            j0 = b * BATCH
            gh = [None] * 4
            sh = [None] * 4
            for j in range(BATCH):
                p = j % 4
                if j >= 4:
                    sh[p].wait()
                gh[p] = pltpu.async_copy(y_sh.at[ridx_v.at[j0 + j]],
                                         rows_v.at[p], gsem.at[p])
                if j >= 1:
                    q = (j - 1) % 4
                    gh[q].wait()
                    sh[q] = pltpu.async_copy(rows_v.at[q],
                                             acc_sh.at[cidx_v.at[j0 + j - 1]],
                                             ssem.at[q], add=True)
            last = (BATCH - 1) % 4
            gh[last].wait()
            sh[last] = pltpu.async_copy(rows_v.at[last],
                                        acc_sh.at[cidx_v.at[j0 + BATCH - 1]],
                                        ssem.at[last], add=True)
            for q in range(4):
                sh[q].wait()
            return 0

        lax.fori_loop(0, NBATCH, batch_body, 0)
        plsc.subcore_barrier()

        pltpu.sync_copy(acc_sh.at[pl.ds(row0, ROWS_PER_TILE), :], stage_v)
        pltpu.sync_copy(stage_v,
                        out_hbm.at[cid, h, pl.ds(row0, ROWS_PER_TILE), :])
        plsc.subcore_barrier()


def _dis_from_parts(degp_ref):
    deg = (degp_ref[0, 0:N, 0:1] + degp_ref[1, 0:N, 0:1]) + 1.0
    return lax.rsqrt(deg)


def _pad_store_halves(yl_ref, yr_ref, y):
    zpad = jnp.zeros((N_PAD - N, HH), jnp.float32)
    yl_ref[pl.ds(0, N), :] = y[:, 0:HH]
    yl_ref[pl.ds(N, N_PAD - N), :] = zpad
    yr_ref[pl.ds(0, N), :] = y[:, HH:H]
    yr_ref[pl.ds(N, N_PAD - N), :] = zpad


def _tc_proj_body(x_ref, win_ref, bin_ref, w1_ref, xw_ref):
    h0 = jnp.dot(x_ref[...], win_ref[...],
                 preferred_element_type=jnp.float32) + bin_ref[...]
    xw_ref[...] = jnp.dot(h0, w1_ref[...],
                          preferred_element_type=jnp.float32)


def _tc_scale_body(xw_ref, degp_ref, yl_ref, yr_ref):
    dis = _dis_from_parts(degp_ref)
    _pad_store_halves(yl_ref, yr_ref, dis * xw_ref[...])


def _graph_norm(o, w, b, ms, eps=1e-5):
    mean = jnp.mean(o, axis=0, keepdims=True)
    out = o - ms * mean
    var = jnp.mean(out * out, axis=0, keepdims=True)
    return w * out / jnp.sqrt(var + eps) + b


def _half_out(a_ref, y_ref, h, dis, b_ref, w_ref, bb_ref, ms_ref):
    sl = pl.ds(h * HH, HH)
    o = dis * (a_ref[0, h, 0:N, :] + a_ref[1, h, 0:N, :] + y_ref[0:N, :]) \
        + b_ref[0:1, sl]
    return _graph_norm(o, w_ref[0:1, sl], bb_ref[0:1, sl], ms_ref[0:1, sl])


def _tc_mid_body(ap_ref, y1l_ref, y1r_ref, degp_ref, b1_ref, w_ref, b_ref,
                 ms_ref, w2_ref, y2l_ref, y2r_ref):
    dis = _dis_from_parts(degp_ref)
    gl = _half_out(ap_ref, y1l_ref, 0, dis, b1_ref, w_ref, b_ref, ms_ref)
    h1l = jnp.where(gl >= 0, gl, 0.2 * gl)
    gr = _half_out(ap_ref, y1r_ref, 1, dis, b1_ref, w_ref, b_ref, ms_ref)
    h1r = jnp.where(gr >= 0, gr, 0.2 * gr)
    y2 = dis * (jnp.dot(h1l, w2_ref[0:HH, :],
                        preferred_element_type=jnp.float32)
                + jnp.dot(h1r, w2_ref[HH:H, :],
                          preferred_element_type=jnp.float32))
    _pad_store_halves(y2l_ref, y2r_ref, y2)


def _tc_post_body(aq_ref, y2l_ref, y2r_ref, degp_ref, b2_ref, w_ref, b_ref,
                  ms_ref, out_ref):
    dis = _dis_from_parts(degp_ref)
    out_ref[:, 0:HH] = _half_out(aq_ref, y2l_ref, 0, dis, b2_ref, w_ref,
                                 b_ref, ms_ref)
    out_ref[:, HH:H] = _half_out(aq_ref, y2r_ref, 1, dis, b2_ref, w_ref,
                                 b_ref, ms_ref)


def kernel(x, edge_index, W_in, b_in, W1, b1, gn1_w, gn1_b, gn1_ms,
           W2, b2, gn2_w, gn2_b, gn2_ms):
    pad = jnp.full((E_PAD - E,), N, jnp.int32)
    row = jnp.concatenate([edge_index[0].astype(jnp.int32), pad])
    col = jnp.concatenate([edge_index[1].astype(jnp.int32), pad])
    row2d = row.reshape(E_PAD // CHUNK, CHUNK)
    col2d = col.reshape(E_PAD // CHUNK, CHUNK)

    degp = _sc_degree(col2d)

    yhalf = jax.ShapeDtypeStruct((N_PAD, HH), jnp.float32)

    xw1 = pl.pallas_call(
        _tc_proj_body,
        out_shape=jax.ShapeDtypeStruct((N, H), jnp.float32),
        compiler_params=_tc_params,
    )(x, W_in, b_in.reshape(1, H), W1)

    y1l, y1r = pl.pallas_call(
        _tc_scale_body,
        out_shape=(yhalf, yhalf),
        compiler_params=_tc_params,
    )(xw1, degp)

    ap = _sc_agg(y1l, y1r, row2d, col2d)

    y2l, y2r = pl.pallas_call(
        _tc_mid_body,
        out_shape=(yhalf, yhalf),
        compiler_params=_tc_params,
    )(ap, y1l, y1r, degp, b1.reshape(1, H), gn1_w.reshape(1, H),
      gn1_b.reshape(1, H), gn1_ms.reshape(1, H), W2)

    aq = _sc_agg(y2l, y2r, row2d, col2d)

    out = pl.pallas_call(
        _tc_post_body,
        out_shape=jax.ShapeDtypeStruct((N, H), jnp.float32),
        compiler_params=_tc_params,
    )(aq, y2l, y2r, degp, b2.reshape(1, H), gn2_w.reshape(1, H),
      gn2_b.reshape(1, H), gn2_ms.reshape(1, H))

    return out


# speedup vs baseline: 1.0311x; 1.0311x over previous
import functools

import jax
import jax.numpy as jnp
from jax import lax
from jax.experimental import pallas as pl
from jax.experimental.pallas import tpu as pltpu
from jax.experimental.pallas import tpu_sc as plsc

N = 10000
E = 320000
D_IN = 128
H = 64
HH = H // 2

NC = 2
NS = 16
NW = NC * NS

CHUNK = 128
E_PAD = 327680
N_PAD = 10016
CPT = E_PAD // CHUNK // NW
BATCH = 16
NBATCH = CPT // BATCH
ROWS_PER_TILE = N_PAD // NS
DEG_W = 16

_mesh = plsc.VectorSubcoreMesh(core_axis_name="c", subcore_axis_name="s")
_params = pltpu.CompilerParams(use_tc_tiling_on_sc=False)
_tc_params = pltpu.CompilerParams(vmem_limit_bytes=100 * 1024 * 1024)


def _worker_id():
    return lax.axis_index("s") * NC + lax.axis_index("c")


def _zero_fill(buf, nrows, ncols):
    zeros = jnp.zeros((16,), jnp.float32)

    def body(r, _):
        for c in range(ncols // 16):
            buf[r, pl.ds(c * 16, 16)] = zeros
        return 0

    lax.fori_loop(0, nrows, body, 0)


@functools.partial(
    pl.kernel,
    mesh=_mesh,
    compiler_params=_params,
    out_type=jax.ShapeDtypeStruct((NC, N_PAD, DEG_W), jnp.float32),
    scratch_types=[
        pltpu.VMEM((CPT, CHUNK), jnp.int32),
        pltpu.VMEM((CHUNK, DEG_W), jnp.float32),
        pltpu.VMEM((ROWS_PER_TILE, DEG_W), jnp.float32),
        pltpu.VMEM_SHARED((N_PAD, DEG_W), jnp.float32),
        pltpu.SemaphoreType.DMA,
    ],
)
def _sc_degree(col_hbm, out_hbm, cidx_v, ones_v, stage_v, acc_sh, sem):
    cid = lax.axis_index("c")
    sid = lax.axis_index("s")
    wid = _worker_id()

    ones = jnp.ones((16,), jnp.float32)

    def fill_ones(r, _):
        ones_v[r, pl.ds(0, 16)] = ones
        return 0

    lax.fori_loop(0, CHUNK, fill_ones, 0)

    _zero_fill(stage_v, ROWS_PER_TILE, DEG_W)
    row0 = sid * ROWS_PER_TILE
    pltpu.sync_copy(stage_v, acc_sh.at[pl.ds(row0, ROWS_PER_TILE), :])
    plsc.subcore_barrier()

    chunk0 = pl.multiple_of(wid * CPT, BATCH)
    pltpu.sync_copy(col_hbm.at[pl.ds(chunk0, CPT), :], cidx_v)

    def batch_body(b, _):
        j0 = b * BATCH
        handles = []
        for j in range(BATCH):
            handles.append(
                pltpu.async_copy(ones_v, acc_sh.at[cidx_v.at[j0 + j]], sem,
                                 add=True))
        for h in handles:
            h.wait()
        return 0

    lax.fori_loop(0, NBATCH, batch_body, 0)
    plsc.subcore_barrier()

    pltpu.sync_copy(acc_sh.at[pl.ds(row0, ROWS_PER_TILE), :], stage_v)
    pltpu.sync_copy(stage_v, out_hbm.at[cid, pl.ds(row0, ROWS_PER_TILE), :])


@functools.partial(
    pl.kernel,
    mesh=_mesh,
    compiler_params=_params,
    out_type=jax.ShapeDtypeStruct((NC, 2, N_PAD, HH), jnp.float32),
    scratch_types=[
        pltpu.VMEM((CPT, CHUNK), jnp.int32),
        pltpu.VMEM((CPT, CHUNK), jnp.int32),
        pltpu.VMEM((4, CHUNK, HH), jnp.float32),
        pltpu.VMEM((ROWS_PER_TILE, HH), jnp.float32),
        pltpu.VMEM_SHARED((N_PAD, HH), jnp.float32),
        pltpu.VMEM_SHARED((N_PAD, HH), jnp.float32),
        pltpu.SemaphoreType.DMA((4,)),
        pltpu.SemaphoreType.DMA((4,)),
    ],
)
def _sc_agg(yl_hbm, yr_hbm, row_hbm, col_hbm, out_hbm,
            ridx_v, cidx_v, rows_v, stage_v, acc_sh, y_sh, gsem, ssem):
    cid = lax.axis_index("c")
    sid = lax.axis_index("s")
    wid = _worker_id()

    row0 = sid * ROWS_PER_TILE
    chunk0 = pl.multiple_of(wid * CPT, BATCH)

    pltpu.sync_copy(row_hbm.at[pl.ds(chunk0, CPT), :], ridx_v)
    pltpu.sync_copy(col_hbm.at[pl.ds(chunk0, CPT), :], cidx_v)
    _zero_fill(stage_v, ROWS_PER_TILE, HH)

    for h, y_hbm in enumerate((yl_hbm, yr_hbm)):
        pltpu.sync_copy(y_hbm.at[pl.ds(row0, ROWS_PER_TILE), :],
                        y_sh.at[pl.ds(row0, ROWS_PER_TILE), :])
        pltpu.sync_copy(stage_v, acc_sh.at[pl.ds(row0, ROWS_PER_TILE), :])
        plsc.subcore_barrier()

        def batch_body(b, _):
            j0 = b * BATCH
            gh = [None] * 4
            sh = [None] * 4
            for j in range(BATCH):
                p = j % 4
                if j >= 4:
                    sh[p].wait()
                gh[p] = pltpu.async_copy(y_sh.at[ridx_v.at[j0 + j]],
                                         rows_v.at[p], gsem.at[p])
                if j >= 1:
                    q = (j - 1) % 4
                    gh[q].wait()
                    sh[q] = pltpu.async_copy(rows_v.at[q],
                                             acc_sh.at[cidx_v.at[j0 + j - 1]],
                                             ssem.at[q], add=True)
            last = (BATCH - 1) % 4
            gh[last].wait()
            sh[last] = pltpu.async_copy(rows_v.at[last],
                                        acc_sh.at[cidx_v.at[j0 + BATCH - 1]],
                                        ssem.at[last], add=True)
            for q in range(4):
                sh[q].wait()
            return 0

        lax.fori_loop(0, NBATCH, batch_body, 0)
        plsc.subcore_barrier()

        pltpu.sync_copy(acc_sh.at[pl.ds(row0, ROWS_PER_TILE), :],
                        out_hbm.at[cid, h, pl.ds(row0, ROWS_PER_TILE), :])
        plsc.subcore_barrier()


def _dis_from_parts(degp_ref):
    deg = (degp_ref[0, 0:N, 0:1] + degp_ref[1, 0:N, 0:1]) + 1.0
    return lax.rsqrt(deg)


def _pad_store_halves(yl_ref, yr_ref, y):
    zpad = jnp.zeros((N_PAD - N, HH), jnp.float32)
    yl_ref[pl.ds(0, N), :] = y[:, 0:HH]
    yl_ref[pl.ds(N, N_PAD - N), :] = zpad
    yr_ref[pl.ds(0, N), :] = y[:, HH:H]
    yr_ref[pl.ds(N, N_PAD - N), :] = zpad


def _tc_proj_body(x_ref, win_ref, bin_ref, w1_ref, xw_ref):
    h0 = jnp.dot(x_ref[...], win_ref[...],
                 preferred_element_type=jnp.float32) + bin_ref[...]
    xw_ref[...] = jnp.dot(h0, w1_ref[...],
                          preferred_element_type=jnp.float32)


def _tc_scale_body(xw_ref, degp_ref, yl_ref, yr_ref):
    dis = _dis_from_parts(degp_ref)
    _pad_store_halves(yl_ref, yr_ref, dis * xw_ref[...])


def _graph_norm(o, w, b, ms, eps=1e-5):
    mean = jnp.mean(o, axis=0, keepdims=True)
    out = o - ms * mean
    var = jnp.mean(out * out, axis=0, keepdims=True)
    return w * out / jnp.sqrt(var + eps) + b


def _half_out(a_ref, y_ref, h, dis, b_ref, w_ref, bb_ref, ms_ref):
    sl = pl.ds(h * HH, HH)
    o = dis * (a_ref[0, h, 0:N, :] + a_ref[1, h, 0:N, :] + y_ref[0:N, :]) \
        + b_ref[0:1, sl]
    return _graph_norm(o, w_ref[0:1, sl], bb_ref[0:1, sl], ms_ref[0:1, sl])


def _tc_mid_body(ap_ref, y1l_ref, y1r_ref, degp_ref, b1_ref, w_ref, b_ref,
                 ms_ref, w2_ref, y2l_ref, y2r_ref):
    dis = _dis_from_parts(degp_ref)
    gl = _half_out(ap_ref, y1l_ref, 0, dis, b1_ref, w_ref, b_ref, ms_ref)
    h1l = jnp.where(gl >= 0, gl, 0.2 * gl)
    gr = _half_out(ap_ref, y1r_ref, 1, dis, b1_ref, w_ref, b_ref, ms_ref)
    h1r = jnp.where(gr >= 0, gr, 0.2 * gr)
    y2 = dis * (jnp.dot(h1l, w2_ref[0:HH, :],
                        preferred_element_type=jnp.float32)
                + jnp.dot(h1r, w2_ref[HH:H, :],
                          preferred_element_type=jnp.float32))
    _pad_store_halves(y2l_ref, y2r_ref, y2)


def _tc_post_body(aq_ref, y2l_ref, y2r_ref, degp_ref, b2_ref, w_ref, b_ref,
                  ms_ref, out_ref):
    dis = _dis_from_parts(degp_ref)
    out_ref[:, 0:HH] = _half_out(aq_ref, y2l_ref, 0, dis, b2_ref, w_ref,
                                 b_ref, ms_ref)
    out_ref[:, HH:H] = _half_out(aq_ref, y2r_ref, 1, dis, b2_ref, w_ref,
                                 b_ref, ms_ref)


def kernel(x, edge_index, W_in, b_in, W1, b1, gn1_w, gn1_b, gn1_ms,
           W2, b2, gn2_w, gn2_b, gn2_ms):
    pad = jnp.full((E_PAD - E,), N, jnp.int32)
    row = jnp.concatenate([edge_index[0].astype(jnp.int32), pad])
    col = jnp.concatenate([edge_index[1].astype(jnp.int32), pad])
    row2d = row.reshape(E_PAD // CHUNK, CHUNK)
    col2d = col.reshape(E_PAD // CHUNK, CHUNK)

    degp = _sc_degree(col2d)

    yhalf = jax.ShapeDtypeStruct((N_PAD, HH), jnp.float32)

    xw1 = pl.pallas_call(
        _tc_proj_body,
        out_shape=jax.ShapeDtypeStruct((N, H), jnp.float32),
        compiler_params=_tc_params,
    )(x, W_in, b_in.reshape(1, H), W1)

    y1l, y1r = pl.pallas_call(
        _tc_scale_body,
        out_shape=(yhalf, yhalf),
        compiler_params=_tc_params,
    )(xw1, degp)

    ap = _sc_agg(y1l, y1r, row2d, col2d)

    y2l, y2r = pl.pallas_call(
        _tc_mid_body,
        out_shape=(yhalf, yhalf),
        compiler_params=_tc_params,
    )(ap, y1l, y1r, degp, b1.reshape(1, H), gn1_w.reshape(1, H),
      gn1_b.reshape(1, H), gn1_ms.reshape(1, H), W2)

    aq = _sc_agg(y2l, y2r, row2d, col2d)

    out = pl.pallas_call(
        _tc_post_body,
        out_shape=jax.ShapeDtypeStruct((N, H), jnp.float32),
        compiler_params=_tc_params,
    )(aq, y2l, y2r, degp, b2.reshape(1, H), gn2_w.reshape(1, H),
      gn2_b.reshape(1, H), gn2_ms.reshape(1, H))

    return out


# speedup vs baseline: 1.0597x; 1.0278x over previous
import functools

import jax
import jax.numpy as jnp
from jax import lax
from jax.experimental import pallas as pl
from jax.experimental.pallas import tpu as pltpu
from jax.experimental.pallas import tpu_sc as plsc

N = 10000
E = 320000
D_IN = 128
H = 64
HH = H // 2

NC = 2
NS = 16
NW = NC * NS

CHUNK = 128
E_PAD = 327680
N_PAD = 10016
CPT = E_PAD // CHUNK // NW
BATCH = 40
NBATCH = CPT // BATCH
DEG_BATCH = 16
DEG_NBATCH = CPT // DEG_BATCH
ROWS_PER_TILE = N_PAD // NS
DEG_W = 16

_mesh = plsc.VectorSubcoreMesh(core_axis_name="c", subcore_axis_name="s")
_params = pltpu.CompilerParams(use_tc_tiling_on_sc=False)
_tc_params = pltpu.CompilerParams(vmem_limit_bytes=100 * 1024 * 1024)


def _worker_id():
    return lax.axis_index("s") * NC + lax.axis_index("c")


def _zero_fill(buf, nrows, ncols):
    zeros = jnp.zeros((16,), jnp.float32)

    def body(r, _):
        for c in range(ncols // 16):
            buf[r, pl.ds(c * 16, 16)] = zeros
        return 0

    lax.fori_loop(0, nrows, body, 0)


@functools.partial(
    pl.kernel,
    mesh=_mesh,
    compiler_params=_params,
    out_type=jax.ShapeDtypeStruct((NC, N_PAD, DEG_W), jnp.float32),
    scratch_types=[
        pltpu.VMEM((CPT, CHUNK), jnp.int32),
        pltpu.VMEM((CHUNK, DEG_W), jnp.float32),
        pltpu.VMEM((ROWS_PER_TILE, DEG_W), jnp.float32),
        pltpu.VMEM_SHARED((N_PAD, DEG_W), jnp.float32),
        pltpu.SemaphoreType.DMA,
    ],
)
def _sc_degree(col_hbm, out_hbm, cidx_v, ones_v, stage_v, acc_sh, sem):
    cid = lax.axis_index("c")
    sid = lax.axis_index("s")
    wid = _worker_id()

    ones = jnp.ones((16,), jnp.float32)

    def fill_ones(r, _):
        ones_v[r, pl.ds(0, 16)] = ones
        return 0

    lax.fori_loop(0, CHUNK, fill_ones, 0)

    _zero_fill(stage_v, ROWS_PER_TILE, DEG_W)
    row0 = sid * ROWS_PER_TILE
    pltpu.sync_copy(stage_v, acc_sh.at[pl.ds(row0, ROWS_PER_TILE), :])
    plsc.subcore_barrier()

    chunk0 = pl.multiple_of(wid * CPT, DEG_BATCH)
    pltpu.sync_copy(col_hbm.at[pl.ds(chunk0, CPT), :], cidx_v)

    def batch_body(b, _):
        j0 = b * DEG_BATCH
        handles = []
        for j in range(DEG_BATCH):
            handles.append(
                pltpu.async_copy(ones_v, acc_sh.at[cidx_v.at[j0 + j]], sem,
                                 add=True))
        for h in handles:
            h.wait()
        return 0

    lax.fori_loop(0, DEG_NBATCH, batch_body, 0)
    plsc.subcore_barrier()

    pltpu.sync_copy(acc_sh.at[pl.ds(row0, ROWS_PER_TILE), :],
                    out_hbm.at[cid, pl.ds(row0, ROWS_PER_TILE), :])


@functools.partial(
    pl.kernel,
    mesh=_mesh,
    compiler_params=_params,
    out_type=jax.ShapeDtypeStruct((NC, 2, N_PAD, HH), jnp.float32),
    scratch_types=[
        pltpu.VMEM((CPT, CHUNK), jnp.int32),
        pltpu.VMEM((CPT, CHUNK), jnp.int32),
        pltpu.VMEM((4, CHUNK, HH), jnp.float32),
        pltpu.VMEM((ROWS_PER_TILE, HH), jnp.float32),
        pltpu.VMEM_SHARED((N_PAD, HH), jnp.float32),
        pltpu.VMEM_SHARED((N_PAD, HH), jnp.float32),
        pltpu.SemaphoreType.DMA((4,)),
        pltpu.SemaphoreType.DMA((4,)),
    ],
)
def _sc_agg(yl_hbm, yr_hbm, row_hbm, col_hbm, out_hbm,
            ridx_v, cidx_v, rows_v, stage_v, acc_sh, y_sh, gsem, ssem):
    cid = lax.axis_index("c")
    sid = lax.axis_index("s")
    wid = _worker_id()

    row0 = sid * ROWS_PER_TILE
    chunk0 = pl.multiple_of(wid * CPT, BATCH)

    pltpu.sync_copy(row_hbm.at[pl.ds(chunk0, CPT), :], ridx_v)
    pltpu.sync_copy(col_hbm.at[pl.ds(chunk0, CPT), :], cidx_v)
    _zero_fill(stage_v, ROWS_PER_TILE, HH)

    for h, y_hbm in enumerate((yl_hbm, yr_hbm)):
        pltpu.sync_copy(y_hbm.at[pl.ds(row0, ROWS_PER_TILE), :],
                        y_sh.at[pl.ds(row0, ROWS_PER_TILE), :])
        pltpu.sync_copy(stage_v, acc_sh.at[pl.ds(row0, ROWS_PER_TILE), :])
        plsc.subcore_barrier()

        def batch_body(b, _):
            j0 = b * BATCH
            gh = [None] * 4
            sh = [None] * 4
            for j in range(BATCH):
                p = j % 4
                if j >= 4:
                    sh[p].wait()
                gh[p] = pltpu.async_copy(y_sh.at[ridx_v.at[j0 + j]],
                                         rows_v.at[p], gsem.at[p])
                if j >= 1:
                    q = (j - 1) % 4
                    gh[q].wait()
                    sh[q] = pltpu.async_copy(rows_v.at[q],
                                             acc_sh.at[cidx_v.at[j0 + j - 1]],
                                             ssem.at[q], add=True)
            last = (BATCH - 1) % 4
            gh[last].wait()
            sh[last] = pltpu.async_copy(rows_v.at[last],
                                        acc_sh.at[cidx_v.at[j0 + BATCH - 1]],
                                        ssem.at[last], add=True)
            for q in range(4):
                sh[q].wait()
            return 0

        lax.fori_loop(0, NBATCH, batch_body, 0)
        plsc.subcore_barrier()

        pltpu.sync_copy(acc_sh.at[pl.ds(row0, ROWS_PER_TILE), :],
                        out_hbm.at[cid, h, pl.ds(row0, ROWS_PER_TILE), :])
        plsc.subcore_barrier()


def _dis_from_parts(degp_ref):
    deg = (degp_ref[0, 0:N, 0:1] + degp_ref[1, 0:N, 0:1]) + 1.0
    return lax.rsqrt(deg)


def _pad_store_halves(yl_ref, yr_ref, y):
    zpad = jnp.zeros((N_PAD - N, HH), jnp.float32)
    yl_ref[pl.ds(0, N), :] = y[:, 0:HH]
    yl_ref[pl.ds(N, N_PAD - N), :] = zpad
    yr_ref[pl.ds(0, N), :] = y[:, HH:H]
    yr_ref[pl.ds(N, N_PAD - N), :] = zpad


def _tc_proj_body(x_ref, win_ref, bin_ref, w1_ref, xw_ref):
    h0 = jnp.dot(x_ref[...], win_ref[...],
                 preferred_element_type=jnp.float32) + bin_ref[...]
    xw_ref[...] = jnp.dot(h0, w1_ref[...],
                          preferred_element_type=jnp.float32)


def _tc_scale_body(xw_ref, degp_ref, yl_ref, yr_ref):
    dis = _dis_from_parts(degp_ref)
    _pad_store_halves(yl_ref, yr_ref, dis * xw_ref[...])


def _graph_norm(o, w, b, ms, eps=1e-5):
    mean = jnp.mean(o, axis=0, keepdims=True)
    out = o - ms * mean
    var = jnp.mean(out * out, axis=0, keepdims=True)
    return w * out / jnp.sqrt(var + eps) + b


def _half_out(a_ref, y_ref, h, dis, b_ref, w_ref, bb_ref, ms_ref):
    sl = pl.ds(h * HH, HH)
    o = dis * (a_ref[0, h, 0:N, :] + a_ref[1, h, 0:N, :] + y_ref[0:N, :]) \
        + b_ref[0:1, sl]
    return _graph_norm(o, w_ref[0:1, sl], bb_ref[0:1, sl], ms_ref[0:1, sl])


def _tc_mid_body(ap_ref, y1l_ref, y1r_ref, degp_ref, b1_ref, w_ref, b_ref,
                 ms_ref, w2_ref, y2l_ref, y2r_ref):
    dis = _dis_from_parts(degp_ref)
    gl = _half_out(ap_ref, y1l_ref, 0, dis, b1_ref, w_ref, b_ref, ms_ref)
    h1l = jnp.where(gl >= 0, gl, 0.2 * gl)
    gr = _half_out(ap_ref, y1r_ref, 1, dis, b1_ref, w_ref, b_ref, ms_ref)
    h1r = jnp.where(gr >= 0, gr, 0.2 * gr)
    y2 = dis * (jnp.dot(h1l, w2_ref[0:HH, :],
                        preferred_element_type=jnp.float32)
                + jnp.dot(h1r, w2_ref[HH:H, :],
                          preferred_element_type=jnp.float32))
    _pad_store_halves(y2l_ref, y2r_ref, y2)


def _tc_post_body(aq_ref, y2l_ref, y2r_ref, degp_ref, b2_ref, w_ref, b_ref,
                  ms_ref, out_ref):
    dis = _dis_from_parts(degp_ref)
    out_ref[:, 0:HH] = _half_out(aq_ref, y2l_ref, 0, dis, b2_ref, w_ref,
                                 b_ref, ms_ref)
    out_ref[:, HH:H] = _half_out(aq_ref, y2r_ref, 1, dis, b2_ref, w_ref,
                                 b_ref, ms_ref)


def kernel(x, edge_index, W_in, b_in, W1, b1, gn1_w, gn1_b, gn1_ms,
           W2, b2, gn2_w, gn2_b, gn2_ms):
    pad = jnp.full((E_PAD - E,), N, jnp.int32)
    row = jnp.concatenate([edge_index[0].astype(jnp.int32), pad])
    col = jnp.concatenate([edge_index[1].astype(jnp.int32), pad])
    row2d = row.reshape(E_PAD // CHUNK, CHUNK)
    col2d = col.reshape(E_PAD // CHUNK, CHUNK)

    degp = _sc_degree(col2d)

    yhalf = jax.ShapeDtypeStruct((N_PAD, HH), jnp.float32)

    xw1 = pl.pallas_call(
        _tc_proj_body,
        out_shape=jax.ShapeDtypeStruct((N, H), jnp.float32),
        compiler_params=_tc_params,
    )(x, W_in, b_in.reshape(1, H), W1)

    y1l, y1r = pl.pallas_call(
        _tc_scale_body,
        out_shape=(yhalf, yhalf),
        compiler_params=_tc_params,
    )(xw1, degp)

    ap = _sc_agg(y1l, y1r, row2d, col2d)

    y2l, y2r = pl.pallas_call(
        _tc_mid_body,
        out_shape=(yhalf, yhalf),
        compiler_params=_tc_params,
    )(ap, y1l, y1r, degp, b1.reshape(1, H), gn1_w.reshape(1, H),
      gn1_b.reshape(1, H), gn1_ms.reshape(1, H), W2)

    aq = _sc_agg(y2l, y2r, row2d, col2d)

    out = pl.pallas_call(
        _tc_post_body,
        out_shape=jax.ShapeDtypeStruct((N, H), jnp.float32),
        compiler_params=_tc_params,
    )(aq, y2l, y2r, degp, b2.reshape(1, H), gn2_w.reshape(1, H),
      gn2_b.reshape(1, H), gn2_ms.reshape(1, H))

    return out
